# baseline (speedup 1.0000x reference)
import jax
import jax.numpy as jnp
from jax import lax
from jax.experimental import pallas as pl
from jax.experimental.pallas import tpu as pltpu
from jax.experimental.pallas import tpu_sc as plsc

NUM_RELATION = 1000
DIM = 64
BATCH = 16384

NC = 2
NS = 16
LANES = 16
NW = NC * NS
BPW = BATCH // NW
NCHUNK = 4
CCOLS = BPW // NCHUNK
GPC = CCOLS // LANES
REP = LANES
DIMR = DIM + REP


def _distmult_kernel(subT_hbm, objT_hbm, rela_hbm, diag_hbm, out_hbm,
                     tab_v, sub_v, obj_v, idx_v, out_v,
                     sem_t, sem_i, sem_s, sem_o):
    wid = lax.axis_index("s") * NC + lax.axis_index("c")
    base = wid * BPW
    lane = lax.iota(jnp.int32, LANES)

    cp_t = pltpu.make_async_copy(diag_hbm, tab_v, sem_t)
    cp_t.start()
    cp_i = pltpu.make_async_copy(rela_hbm.at[pl.ds(base, BPW)], idx_v, sem_i)
    cp_i.start()

    def start_chunk(c):
        b = c % 2
        cb = base + c * CCOLS
        cps = [
            pltpu.make_async_copy(
                subT_hbm.at[:, pl.ds(cb, CCOLS)],
                sub_v.at[b, pl.ds(0, DIM)], sem_s),
            pltpu.make_async_copy(
                subT_hbm.at[pl.ds(0, REP), pl.ds(cb, CCOLS)],
                sub_v.at[b, pl.ds(DIM, REP)], sem_s),
            pltpu.make_async_copy(
                objT_hbm.at[:, pl.ds(cb, CCOLS)],
                obj_v.at[b, pl.ds(0, DIM)], sem_o),
            pltpu.make_async_copy(
                objT_hbm.at[pl.ds(0, REP), pl.ds(cb, CCOLS)],
                obj_v.at[b, pl.ds(DIM, REP)], sem_o),
        ]
        for cp in cps:
            cp.start()
        return cps

    cps = start_chunk(0)
    cp_t.wait()
    cp_i.wait()

    for c in range(NCHUNK):
        for cp in cps:
            cp.wait()
        if c + 1 < NCHUNK:
            cps = start_chunk(c + 1)

        b = c % 2
        sref = sub_v.at[b]
        oref = obj_v.at[b]

        @plsc.parallel_loop(0, GPC)
        def body(g):
            gb = g * LANES
            rv = idx_v[pl.ds(c * CCOLS + gb, LANES)]
            gr = rv * DIM
            col = gb + lane
            accs = [None, None]
            for d in range(DIM):
                drow = lane + d
                s = plsc.load_gather(sref, [drow, col])
                dd = plsc.load_gather(tab_v, [gr + ((lane + d) & (DIM - 1))])
                o = plsc.load_gather(oref, [drow, col])
                p = s * dd * o
                k = d % 2
                accs[k] = p if accs[k] is None else accs[k] + p
            out_v[pl.ds(c * CCOLS + gb, LANES)] = accs[0] + accs[1]

    pltpu.sync_copy(out_v, out_hbm.at[pl.ds(base, BPW)])


@jax.jit
def kernel(sub_embed, obj_embed, rela, diag):
    mesh = plsc.VectorSubcoreMesh(core_axis_name="c", subcore_axis_name="s")
    run = pl.kernel(
        _distmult_kernel,
        out_type=jax.ShapeDtypeStruct((BATCH,), jnp.float32),
        mesh=mesh,
        scratch_types=[
            pltpu.VMEM((NUM_RELATION * DIM,), jnp.float32),
            pltpu.VMEM((2, DIMR, CCOLS), jnp.float32),
            pltpu.VMEM((2, DIMR, CCOLS), jnp.float32),
            pltpu.VMEM((BPW,), jnp.int32),
            pltpu.VMEM((BPW,), jnp.float32),
            pltpu.SemaphoreType.DMA,
            pltpu.SemaphoreType.DMA,
            pltpu.SemaphoreType.DMA,
            pltpu.SemaphoreType.DMA,
        ],
        compiler_params=pltpu.CompilerParams(needs_layout_passes=False),
    )
    return run(jnp.swapaxes(sub_embed, 0, 1), jnp.swapaxes(obj_embed, 0, 1),
               rela.astype(jnp.int32), diag.reshape(-1))
